# Initial kernel scaffold; baseline (speedup 1.0000x reference)
import jax
import jax.numpy as jnp
from jax.experimental import pallas as pl


def kernel(feat_supp, feat_curr, flow, Wq, bq, Wk, bk, Wv, bv):
    raise NotImplementedError("write your pallas kernel here")


# speedup vs baseline: 1473.7068x; 1473.7068x over previous
import functools
import math

import jax
import jax.numpy as jnp
import numpy as np
from jax import lax
from jax.experimental import pallas as pl
from jax.experimental.pallas import tpu as pltpu
from jax.experimental.pallas import tpu_sc as plsc

N, C, H, W = 2, 256, 128, 128
HW = H * W
DIM = 256
PE_DIM = 256
HEADS = 8
HEAD_DIM = DIM // HEADS
WS = 2
NWP = WS * WS
TEMP = 10000.0
SCALE = HEAD_DIM ** (-0.5)

BPA = 512
NBA = HW // BPA
BPB = 256
NBB = HW // BPB

NR = N * NWP * HW
NWORK = 32
PER_W = NR // NWORK
CH = 128
NIT = PER_W // CH


def _pos_bias_padded():
    npf = PE_DIM // 2
    eps = 1e-6
    scale = 2 * math.pi
    emb = np.arange(WS, dtype=np.float64) / (WS - 1 + eps) * scale
    dim_t = TEMP ** (2 * (np.arange(npf) // 2) / npf)
    yy, xx = np.meshgrid(emb, emb, indexing="ij")

    def interleave(a):
        out = np.empty_like(a)
        out[..., 0::2] = np.sin(a[..., 0::2])
        out[..., 1::2] = np.cos(a[..., 1::2])
        return out

    posy = interleave(yy[..., None] / dim_t)
    posx = interleave(xx[..., None] / dim_t)
    pos = np.concatenate([posy, posx], axis=-1).reshape(NWP, PE_DIM)
    return np.concatenate([pos, np.zeros((8 - NWP, PE_DIM))], 0).astype(np.float32)


_PB8 = _pos_bias_padded()


def _proj_body(fs_ref, fl_ref, wk_ref, wv_ref, kv_ref, idx_ref):
    n_i = pl.program_id(0)
    b_i = pl.program_id(1)
    fs = fs_ref[0]
    kf = lax.dot_general(fs, wk_ref[...], (((0,), (1,)), ((), ())),
                         preferred_element_type=jnp.float32)
    vf = lax.dot_general(fs, wv_ref[...], (((0,), (1,)), ((), ())),
                         preferred_element_type=jnp.float32)
    kv_ref[0, :, :DIM] = kf
    kv_ref[0, :, DIM:] = vf

    fl = fl_ref[0, 0]
    p = b_i * BPA + lax.broadcasted_iota(jnp.int32, (1, BPA), 1)
    yi = p // W
    xi = p - yi * W
    gx = xi.astype(jnp.float32) + fl[0:1, :]
    gy = yi.astype(jnp.float32) + fl[1:2, :]
    x0 = jnp.floor(gx).astype(jnp.int32)
    y0 = jnp.floor(gy).astype(jnp.int32)
    rows = []
    for dy in (0, 1):
        hy = jnp.clip(y0 + dy, 0, H - 1)
        for dx in (0, 1):
            wx = jnp.clip(x0 + dx, 0, W - 1)
            rows.append(wx + W * hy + n_i * HW)
    idx_ref[0] = jnp.concatenate(rows + rows, axis=0)


def _proj_call(fs, fl_a, Wk, Wv):
    return pl.pallas_call(
        _proj_body,
        grid=(N, NBA),
        in_specs=[
            pl.BlockSpec((1, C, BPA), lambda n, b: (n, 0, b)),
            pl.BlockSpec((1, 1, 2, BPA), lambda n, b: (n, b, 0, 0)),
            pl.BlockSpec((DIM, DIM), lambda n, b: (0, 0)),
            pl.BlockSpec((DIM, DIM), lambda n, b: (0, 0)),
        ],
        out_specs=[
            pl.BlockSpec((1, BPA, 2 * DIM), lambda n, b: (n, b, 0)),
            pl.BlockSpec((1, 8, BPA), lambda n, b: (n, 0, b)),
        ],
        out_shape=[
            jax.ShapeDtypeStruct((N, HW, 2 * DIM), jnp.float32),
            jax.ShapeDtypeStruct((N, 8, HW), jnp.int32),
        ],
    )(fs, fl_a, Wk, Wv)


def _sc_gather(table, idx):
    mesh = plsc.VectorSubcoreMesh(core_axis_name="c", subcore_axis_name="s",
                                  num_cores=2, num_subcores=16)

    @functools.partial(
        pl.kernel,
        mesh=mesh,
        out_type=jax.ShapeDtypeStruct((NR, 2 * DIM), jnp.float32),
        scratch_types=[
            pltpu.VMEM((CH,), jnp.int32),
            pltpu.VMEM((CH, 2 * DIM), jnp.float32),
            pltpu.SemaphoreType.DMA,
        ],
    )
    def gather_kernel(table_hbm, idx_hbm, out_hbm, idx_v, rows_v, sem):
        wid = lax.axis_index("s") * 2 + lax.axis_index("c")
        base = wid * PER_W

        def body(i, carry):
            off = base + i * CH
            pltpu.sync_copy(idx_hbm.at[pl.ds(off, CH)], idx_v)
            pltpu.async_copy(table_hbm.at[idx_v], rows_v, sem).wait()
            pltpu.sync_copy(rows_v, out_hbm.at[pl.ds(off, CH)])
            return carry

        lax.fori_loop(0, NIT, body, 0)

    return gather_kernel(table, idx)


def _attn_body(g_ref, fc_ref, fl_ref, pb_ref, wq_ref, wk_ref, wv_ref, b3_ref,
               out_ref):
    fl = fl_ref[0, 0]
    fx = fl[:, 0:1]
    fy = fl[:, 1:2]
    dx = fx - jnp.floor(fx)
    dy = fy - jnp.floor(fy)

    d = lax.broadcasted_iota(jnp.int32, (BPB, PE_DIM // 2), 1)
    inv_t = jnp.exp((d // 2).astype(jnp.float32) *
                    (-math.log(TEMP) / (PE_DIM // 4)))
    even = (d % 2) == 0
    cc = 2 * math.pi / (WS + 1e-6)
    ay = dy * cc * inv_t
    ax = dx * cc * inv_t
    pey = jnp.where(even, jnp.sin(ay), jnp.cos(ay))
    pex = jnp.where(even, jnp.sin(ax), jnp.cos(ax))
    pe = jnp.concatenate([pey, pex], axis=1)

    wq = wq_ref[...]
    q = lax.dot_general(fc_ref[0], wq, (((0,), (1,)), ((), ())),
                        preferred_element_type=jnp.float32)
    q = q + lax.dot_general(pe, wq, (((1,), (1,)), ((), ())),
                            preferred_element_type=jnp.float32)
    q = (q + b3_ref[0:1, :]) * SCALE

    kpe = lax.dot_general(pb_ref[...], wk_ref[...], (((1,), (1,)), ((), ())),
                          preferred_element_type=jnp.float32) + b3_ref[1:2, :]
    vpe = lax.dot_general(pb_ref[...], wv_ref[...], (((1,), (1,)), ((), ())),
                          preferred_element_type=jnp.float32) + b3_ref[2:3, :]

    hsel = (lax.broadcasted_iota(jnp.int32, (DIM, HEADS), 0) // HEAD_DIM ==
            lax.broadcasted_iota(jnp.int32, (DIM, HEADS), 1))
    m = hsel.astype(jnp.float32)

    logits = []
    for j in range(NWP):
        kj = g_ref[0, j, :, :DIM] + kpe[j:j + 1, :]
        logits.append(lax.dot_general(q * kj, m, (((1,), (0,)), ((), ())),
                                      preferred_element_type=jnp.float32))
    mx = jnp.maximum(jnp.maximum(logits[0], logits[1]),
                     jnp.maximum(logits[2], logits[3]))
    es = [jnp.exp(l - mx) for l in logits]
    inv = 1.0 / (es[0] + es[1] + es[2] + es[3])
    acc = jnp.zeros((BPB, DIM), jnp.float32)
    for j in range(NWP):
        wj = es[j] * inv
        wb = lax.dot_general(wj, m, (((1,), (1,)), ((), ())),
                             preferred_element_type=jnp.float32)
        acc = acc + wb * (g_ref[0, j, :, DIM:] + vpe[j:j + 1, :])
    out_ref[0] = acc


def _attn_call(g4, fc, fl_b, pb8, Wq, Wk, Wv, b3):
    return pl.pallas_call(
        _attn_body,
        grid=(N, NBB),
        in_specs=[
            pl.BlockSpec((1, NWP, BPB, 2 * DIM), lambda n, b: (n, 0, b, 0)),
            pl.BlockSpec((1, C, BPB), lambda n, b: (n, 0, b)),
            pl.BlockSpec((1, 1, BPB, 2), lambda n, b: (n, b, 0, 0)),
            pl.BlockSpec((8, PE_DIM), lambda n, b: (0, 0)),
            pl.BlockSpec((DIM, DIM), lambda n, b: (0, 0)),
            pl.BlockSpec((DIM, DIM), lambda n, b: (0, 0)),
            pl.BlockSpec((DIM, DIM), lambda n, b: (0, 0)),
            pl.BlockSpec((8, DIM), lambda n, b: (0, 0)),
        ],
        out_specs=pl.BlockSpec((1, BPB, DIM), lambda n, b: (n, b, 0)),
        out_shape=jax.ShapeDtypeStruct((N, HW, DIM), jnp.float32),
    )(g4, fc, fl_b, pb8, Wq, Wk, Wv, b3)


def kernel(feat_supp, feat_curr, flow, Wq, bq, Wk, bk, Wv, bv):
    fs = feat_supp.reshape(N, C, HW)
    fc = feat_curr.reshape(N, C, HW)
    flf = flow.reshape(N, HW, 2)
    fl_a = jnp.transpose(flf.reshape(N, NBA, BPA, 2), (0, 1, 3, 2))
    fl_b = flf.reshape(N, NBB, BPB, 2)
    b3 = jnp.concatenate(
        [bq[None], bk[None], bv[None], jnp.zeros((5, DIM), jnp.float32)], 0)
    pb8 = jnp.asarray(_PB8)

    kv, idx8 = _proj_call(fs, fl_a, Wk, Wv)
    idxf = idx8[:, :NWP, :].reshape(NR)
    gathered = _sc_gather(kv.reshape(N * HW, 2 * DIM), idxf)
    g4 = gathered.reshape(N, NWP, HW, 2 * DIM)

    out = _attn_call(g4, fc, fl_b, pb8, Wq, Wk, Wv, b3)
    return out.reshape(N, H, W, DIM).transpose(0, 3, 1, 2)


# speedup vs baseline: 1782.7585x; 1.2097x over previous
import functools
import math

import jax
import jax.numpy as jnp
import numpy as np
from jax import lax
from jax.experimental import pallas as pl
from jax.experimental.pallas import tpu as pltpu
from jax.experimental.pallas import tpu_sc as plsc

N, C, H, W = 2, 256, 128, 128
HW = H * W
DIM = 256
PE_DIM = 256
HEADS = 8
HEAD_DIM = DIM // HEADS
WS = 2
NWP = WS * WS
TEMP = 10000.0
SCALE = HEAD_DIM ** (-0.5)

BPA = 512
NBA = HW // BPA
BPB = 512
NBB = HW // BPB

NWORK = 32
PER_W = NWP * N * HW // NWORK
CH = 128
NIT = PER_W // CH

_HI = -65536


def _pos_bias_padded():
    npf = PE_DIM // 2
    eps = 1e-6
    scale = 2 * math.pi
    emb = np.arange(WS, dtype=np.float64) / (WS - 1 + eps) * scale
    dim_t = TEMP ** (2 * (np.arange(npf) // 2) / npf)
    yy, xx = np.meshgrid(emb, emb, indexing="ij")

    def interleave(a):
        out = np.empty_like(a)
        out[..., 0::2] = np.sin(a[..., 0::2])
        out[..., 1::2] = np.cos(a[..., 1::2])
        return out

    posy = interleave(yy[..., None] / dim_t)
    posx = interleave(xx[..., None] / dim_t)
    pos = np.concatenate([posy, posx], axis=-1).reshape(NWP, PE_DIM)
    return np.concatenate([pos, np.zeros((8 - NWP, PE_DIM))], 0).astype(np.float32)


_PB8 = _pos_bias_padded()


def _round_bf16_bits(x):
    b = lax.bitcast_convert_type(x, jnp.int32)
    return b + 0x7FFF + ((b >> 16) & 1)


def _proj_body(fs_ref, fl_ref, wk_ref, wv_ref, kv_ref, idx_ref):
    n_i = pl.program_id(0)
    b_i = pl.program_id(1)
    fs = fs_ref[0]
    kf = lax.dot_general(fs, wk_ref[...], (((0,), (1,)), ((), ())),
                         preferred_element_type=jnp.float32)
    vf = lax.dot_general(fs, wv_ref[...], (((0,), (1,)), ((), ())),
                         preferred_element_type=jnp.float32)
    kr = _round_bf16_bits(kf)
    vr = _round_bf16_bits(vf)
    kv_ref[0] = (vr & _HI) | ((kr >> 16) & 0xFFFF)

    fl = fl_ref[0, 0]
    p = b_i * BPA + lax.broadcasted_iota(jnp.int32, (1, BPA), 1)
    yi = p // W
    xi = p - yi * W
    gx = xi.astype(jnp.float32) + fl[0:1, :]
    gy = yi.astype(jnp.float32) + fl[1:2, :]
    x0 = jnp.floor(gx).astype(jnp.int32)
    y0 = jnp.floor(gy).astype(jnp.int32)
    rows = []
    for dy in (0, 1):
        hy = jnp.clip(y0 + dy, 0, H - 1)
        for dx in (0, 1):
            wx = jnp.clip(x0 + dx, 0, W - 1)
            rows.append(wx + W * hy + n_i * HW)
    idx_ref[0] = jnp.concatenate(rows + rows, axis=0)


def _proj_call(fs, fl_a, Wk, Wv):
    return pl.pallas_call(
        _proj_body,
        grid=(N, NBA),
        in_specs=[
            pl.BlockSpec((1, C, BPA), lambda n, b: (n, 0, b)),
            pl.BlockSpec((1, 1, 2, BPA), lambda n, b: (n, b, 0, 0)),
            pl.BlockSpec((DIM, DIM), lambda n, b: (0, 0)),
            pl.BlockSpec((DIM, DIM), lambda n, b: (0, 0)),
        ],
        out_specs=[
            pl.BlockSpec((1, BPA, DIM), lambda n, b: (n, b, 0)),
            pl.BlockSpec((1, 8, BPA), lambda n, b: (n, 0, b)),
        ],
        out_shape=[
            jax.ShapeDtypeStruct((N, HW, DIM), jnp.int32),
            jax.ShapeDtypeStruct((N, 8, HW), jnp.int32),
        ],
    )(fs, fl_a, Wk, Wv)


def _sc_gather(table, idx8):
    mesh = plsc.VectorSubcoreMesh(core_axis_name="c", subcore_axis_name="s",
                                  num_cores=2, num_subcores=16)

    @functools.partial(
        pl.kernel,
        mesh=mesh,
        out_type=jax.ShapeDtypeStruct((N, NWP, HW, DIM), jnp.int32),
        scratch_types=[
            pltpu.VMEM((CH,), jnp.int32),
            pltpu.VMEM((CH, DIM), jnp.int32),
            pltpu.SemaphoreType.DMA,
        ],
    )
    def gather_kernel(table_hbm, idx_hbm, out_hbm, idx_v, rows_v, sem):
        wid = lax.axis_index("s") * 2 + lax.axis_index("c")
        n = wid // 16
        rem = wid - n * 16
        j = rem // 4
        p0 = (rem - j * 4) * PER_W

        def body(i, carry):
            off = p0 + i * CH
            pltpu.sync_copy(idx_hbm.at[n, j, pl.ds(off, CH)], idx_v)
            pltpu.async_copy(table_hbm.at[idx_v], rows_v, sem).wait()
            pltpu.sync_copy(rows_v, out_hbm.at[n, j, pl.ds(off, CH)])
            return carry

        lax.fori_loop(0, NIT, body, 0)

    return gather_kernel(table, idx8)


def _attn_body(g_ref, fc_ref, fl_ref, pb_ref, wq_ref, wk_ref, wv_ref, b3_ref,
               out_ref):
    fl = fl_ref[0, 0]
    fx = fl[:, 0:1]
    fy = fl[:, 1:2]
    dx = fx - jnp.floor(fx)
    dy = fy - jnp.floor(fy)

    d = lax.broadcasted_iota(jnp.int32, (BPB, PE_DIM // 2), 1)
    inv_t = jnp.exp((d // 2).astype(jnp.float32) *
                    (-math.log(TEMP) / (PE_DIM // 4)))
    even = (d % 2) == 0
    cc = 2 * math.pi / (WS + 1e-6)
    ay = dy * cc * inv_t
    ax = dx * cc * inv_t
    pey = jnp.where(even, jnp.sin(ay), jnp.cos(ay))
    pex = jnp.where(even, jnp.sin(ax), jnp.cos(ax))
    pe = jnp.concatenate([pey, pex], axis=1)

    q = lax.dot_general(fc_ref[0], wq_ref[...], (((0,), (1,)), ((), ())),
                        preferred_element_type=jnp.float32)
    q = q + lax.dot_general(pe, wq_ref[...], (((1,), (1,)), ((), ())),
                            preferred_element_type=jnp.float32)
    q = (q + b3_ref[0:1, :]) * SCALE

    kpe = lax.dot_general(pb_ref[...], wk_ref[...], (((1,), (1,)), ((), ())),
                          preferred_element_type=jnp.float32) + b3_ref[1:2, :]
    vpe = lax.dot_general(pb_ref[...], wv_ref[...], (((1,), (1,)), ((), ())),
                          preferred_element_type=jnp.float32) + b3_ref[2:3, :]

    hsel = (lax.broadcasted_iota(jnp.int32, (DIM, HEADS), 0) // HEAD_DIM ==
            lax.broadcasted_iota(jnp.int32, (DIM, HEADS), 1))
    m = hsel.astype(jnp.float32)

    gs = [g_ref[0, j] for j in range(NWP)]
    logits = []
    for j in range(NWP):
        kj = lax.bitcast_convert_type(gs[j] << 16, jnp.float32) + kpe[j:j + 1, :]
        logits.append(lax.dot_general(q * kj, m, (((1,), (0,)), ((), ())),
                                      preferred_element_type=jnp.float32))
    mx = jnp.maximum(jnp.maximum(logits[0], logits[1]),
                     jnp.maximum(logits[2], logits[3]))
    es = [jnp.exp(l - mx) for l in logits]
    inv = 1.0 / (es[0] + es[1] + es[2] + es[3])
    acc = jnp.zeros((BPB, DIM), jnp.float32)
    for j in range(NWP):
        wj = es[j] * inv
        wb = lax.dot_general(wj, m, (((1,), (1,)), ((), ())),
                             preferred_element_type=jnp.float32)
        vj = lax.bitcast_convert_type(gs[j] & _HI, jnp.float32) + vpe[j:j + 1, :]
        acc = acc + wb * vj
    out_ref[0] = acc


def _attn_call(g4, fc, fl_b, pb8, Wq, Wk, Wv, b3):
    return pl.pallas_call(
        _attn_body,
        grid=(N, NBB),
        in_specs=[
            pl.BlockSpec((1, NWP, BPB, DIM), lambda n, b: (n, 0, b, 0)),
            pl.BlockSpec((1, C, BPB), lambda n, b: (n, 0, b)),
            pl.BlockSpec((1, 1, BPB, 2), lambda n, b: (n, b, 0, 0)),
            pl.BlockSpec((8, PE_DIM), lambda n, b: (0, 0)),
            pl.BlockSpec((DIM, DIM), lambda n, b: (0, 0)),
            pl.BlockSpec((DIM, DIM), lambda n, b: (0, 0)),
            pl.BlockSpec((DIM, DIM), lambda n, b: (0, 0)),
            pl.BlockSpec((8, DIM), lambda n, b: (0, 0)),
        ],
        out_specs=pl.BlockSpec((1, BPB, DIM), lambda n, b: (n, b, 0)),
        out_shape=jax.ShapeDtypeStruct((N, HW, DIM), jnp.float32),
    )(g4, fc, fl_b, pb8, Wq, Wk, Wv, b3)


def kernel(feat_supp, feat_curr, flow, Wq, bq, Wk, bk, Wv, bv):
    fs = feat_supp.reshape(N, C, HW)
    fc = feat_curr.reshape(N, C, HW)
    flf = flow.reshape(N, HW, 2)
    fl_a = jnp.transpose(flf.reshape(N, NBA, BPA, 2), (0, 1, 3, 2))
    fl_b = flf.reshape(N, NBB, BPB, 2)
    b3 = jnp.concatenate(
        [bq[None], bk[None], bv[None], jnp.zeros((5, DIM), jnp.float32)], 0)
    pb8 = jnp.asarray(_PB8)

    kv, idx8 = _proj_call(fs, fl_a, Wk, Wv)
    g4 = _sc_gather(kv.reshape(N * HW, DIM), idx8)

    out = _attn_call(g4, fc, fl_b, pb8, Wq, Wk, Wv, b3)
    return out.reshape(N, H, W, DIM).transpose(0, 3, 1, 2)
